# baseline (speedup 1.0000x reference)
import jax
import jax.numpy as jnp
from jax import lax
from jax.experimental import pallas as pl
from jax.experimental.pallas import tpu as pltpu
from jax.experimental.pallas import tpu_sc as plsc

_NC = 2
_NS = 16
_NW = _NC * _NS
_B = 128
_GB = 80

_MESH = plsc.VectorSubcoreMesh(core_axis_name="c", subcore_axis_name="s")


def _wid():
    return lax.axis_index("s") * _NC + lax.axis_index("c")


def _sc_embed_deg(emb, ids_p, col3d, zeros_n, nbc0, nbc1):
    V, D = emb.shape
    Np = ids_p.shape[0]
    nb_e = col3d.shape[1]
    rows_w = Np // _NW
    rpt = Np // _NS

    def body(emb_h, ids_h, col_h, zn_h, h_out, pdeg0_out, pdeg1_out,
             idx_v, rows_v, cidx_v, ones_v, dacc, sem):
        c = lax.axis_index("c")
        s = lax.axis_index("s")
        w = _wid()
        for k in range(rows_w // _GB):
            base = w * rows_w + k * _GB
            pltpu.sync_copy(ids_h.at[pl.ds(base, _GB)], idx_v)
            pltpu.async_copy(emb_h.at[idx_v], rows_v, sem).wait()
            pltpu.sync_copy(rows_v, h_out.at[pl.ds(base, _GB)])
        t0 = s * rpt
        pltpu.sync_copy(zn_h.at[pl.ds(t0, rpt)], dacc.at[pl.ds(t0, rpt)])
        for j in range(_B // 16):
            ones_v[pl.ds(16 * j, 16)] = jnp.ones((16,), jnp.float32)
        pltpu.sync_copy(col_h.at[w], cidx_v)
        plsc.subcore_barrier()

        def step(j, carry):
            pltpu.sync_copy(ones_v, dacc.at[cidx_v.at[j]], add=True)
            return carry

        @pl.when(c == 0)
        def _():
            lax.fori_loop(0, nbc0, step, 0)

        @pl.when(c != 0)
        def _():
            lax.fori_loop(0, nbc1, step, 0)

        plsc.subcore_barrier()

        @pl.when(c == 0)
        def _():
            pltpu.sync_copy(dacc.at[pl.ds(t0, rpt)],
                            pdeg0_out.at[pl.ds(t0, rpt)])

        @pl.when(c != 0)
        def _():
            pltpu.sync_copy(dacc.at[pl.ds(t0, rpt)],
                            pdeg1_out.at[pl.ds(t0, rpt)])

    f = pl.kernel(
        body,
        out_type=(jax.ShapeDtypeStruct((Np, D), jnp.float32),
                  jax.ShapeDtypeStruct((Np,), jnp.float32),
                  jax.ShapeDtypeStruct((Np,), jnp.float32)),
        mesh=_MESH,
        scratch_types=[
            pltpu.VMEM((_GB,), jnp.int32),
            pltpu.VMEM((_GB, D), jnp.float32),
            pltpu.VMEM((nb_e, _B), jnp.int32),
            pltpu.VMEM((_B,), jnp.float32),
            pltpu.VMEM_SHARED((Np,), jnp.float32),
            pltpu.SemaphoreType.DMA,
        ],
    )
    return f(emb, ids_p, col3d, zeros_n)


def _sc_edge_agg(zt, row3d, col3d, zeros_nd, nbc0, nbc1):
    Np, D = zt.shape
    nb = row3d.shape[1]
    rpt = Np // _NS

    def body(zt_h, row_h, col_h, zn_h, p_out,
             ridx, cidx, rows_v, acc, sem):
        c = lax.axis_index("c")
        s = lax.axis_index("s")
        w = _wid()
        t0 = s * rpt

        @pl.when(c == 0)
        def _():
            pltpu.sync_copy(zt_h.at[pl.ds(t0, rpt)], acc.at[pl.ds(t0, rpt)])

        @pl.when(c != 0)
        def _():
            pltpu.sync_copy(zn_h.at[pl.ds(t0, rpt)], acc.at[pl.ds(t0, rpt)])

        pltpu.sync_copy(row_h.at[w], ridx)
        pltpu.sync_copy(col_h.at[w], cidx)
        plsc.subcore_barrier()

        def step(j, carry):
            pltpu.async_copy(zt_h.at[ridx.at[j]], rows_v, sem).wait()
            pltpu.sync_copy(rows_v, acc.at[cidx.at[j]], add=True)
            return carry

        @pl.when(c == 0)
        def _():
            lax.fori_loop(0, nbc0, step, 0)

        @pl.when(c != 0)
        def _():
            lax.fori_loop(0, nbc1, step, 0)

        plsc.subcore_barrier()
        pltpu.sync_copy(acc.at[pl.ds(t0, rpt)], p_out.at[c, pl.ds(t0, rpt)])

    f = pl.kernel(
        body,
        out_type=jax.ShapeDtypeStruct((_NC, Np, D), jnp.float32),
        mesh=_MESH,
        scratch_types=[
            pltpu.VMEM((nb, _B), jnp.int32),
            pltpu.VMEM((nb, _B), jnp.int32),
            pltpu.VMEM((_B, D), jnp.float32),
            pltpu.VMEM_SHARED((Np, D), jnp.float32),
            pltpu.SemaphoreType.DMA,
        ],
    )
    return f(zt, row3d, col3d, zeros_nd)


def _sc_take(h, tgt):
    Np, D = h.shape
    T = tgt.shape[0]
    rw = T // _NW

    def body(h_h, t_h, o_h, idx_v, rows_v, sem):
        base = _wid() * rw
        pltpu.sync_copy(t_h.at[pl.ds(base, rw)], idx_v)
        pltpu.async_copy(h_h.at[idx_v], rows_v, sem).wait()
        pltpu.sync_copy(rows_v, o_h.at[pl.ds(base, rw)])

    f = pl.kernel(
        body,
        out_type=jax.ShapeDtypeStruct((T, D), jnp.float32),
        mesh=_MESH,
        scratch_types=[
            pltpu.VMEM((rw,), jnp.int32),
            pltpu.VMEM((rw, D), jnp.float32),
            pltpu.SemaphoreType.DMA,
        ],
    )
    return f(h, tgt)


_BLK = 1024


def _dinv_blk(d0_ref, d1_ref):
    deg = d0_ref[...] + d1_ref[...] + 1.0
    return lax.rsqrt(deg)


def _mm_scale_body(h_ref, w_ref, d0_ref, d1_ref, o_ref):
    dinv = _dinv_blk(d0_ref, d1_ref)
    z = jnp.dot(h_ref[...], w_ref[...], preferred_element_type=jnp.float32)
    o_ref[...] = z * dinv


def _ln_relu(t, g, b):
    m = jnp.mean(t, axis=-1, keepdims=True)
    cdev = t - m
    v = jnp.mean(cdev * cdev, axis=-1, keepdims=True)
    return jnp.maximum(cdev * lax.rsqrt(v + 1e-5) * g + b, 0.0)


def _agg_ln_mm_body(p_ref, d0_ref, d1_ref, b_ref, g_ref, bb_ref, w_ref,
                    o_ref):
    dinv = _dinv_blk(d0_ref, d1_ref)
    t = (p_ref[0] + p_ref[1]) * dinv + b_ref[...]
    hr = _ln_relu(t, g_ref[...], bb_ref[...])
    o_ref[...] = jnp.dot(hr, w_ref[...],
                         preferred_element_type=jnp.float32) * dinv


def _agg_ln_body(p_ref, d0_ref, d1_ref, b_ref, g_ref, bb_ref, o_ref):
    dinv = _dinv_blk(d0_ref, d1_ref)
    t = (p_ref[0] + p_ref[1]) * dinv + b_ref[...]
    o_ref[...] = _ln_relu(t, g_ref[...], bb_ref[...])


def _mlp_body(t_ref, w0, b0, g0, bb0, w1, b1, g1, bb1, wo, bo, o_ref):
    t = t_ref[...]
    for w, b, g, bb in ((w0, b0, g0, bb0), (w1, b1, g1, bb1)):
        z = jnp.dot(t, w[...], preferred_element_type=jnp.float32) + b[...]
        t = _ln_relu(z, g[...], bb[...])
    o_ref[...] = jnp.dot(t, wo[...], preferred_element_type=jnp.float32) \
        + bo[...]


def _row_spec(d):
    return pl.BlockSpec((_BLK, d), lambda i: (i, 0))


def _full_spec(shape):
    nd = len(shape)
    return pl.BlockSpec(shape, lambda i, _n=nd: (0,) * _n)


def _tc_mm_scale(h, w, d0, d1):
    Np, D = h.shape
    return pl.pallas_call(
        _mm_scale_body,
        grid=(Np // _BLK,),
        in_specs=[_row_spec(D), _full_spec((D, D)),
                  _row_spec(1), _row_spec(1)],
        out_specs=_row_spec(D),
        out_shape=jax.ShapeDtypeStruct((Np, D), jnp.float32),
    )(h, w, d0, d1)


def _tc_agg_ln_mm(p, d0, d1, b, g, bb, w_next):
    _, Np, D = p.shape
    return pl.pallas_call(
        _agg_ln_mm_body,
        grid=(Np // _BLK,),
        in_specs=[pl.BlockSpec((_NC, _BLK, D), lambda i: (0, i, 0)),
                  _row_spec(1), _row_spec(1),
                  _full_spec((1, D)), _full_spec((1, D)), _full_spec((1, D)),
                  _full_spec((D, D))],
        out_specs=_row_spec(D),
        out_shape=jax.ShapeDtypeStruct((Np, D), jnp.float32),
    )(p, d0, d1, b, g, bb, w_next)


def _tc_agg_ln(p, d0, d1, b, g, bb):
    _, Np, D = p.shape
    return pl.pallas_call(
        _agg_ln_body,
        grid=(Np // _BLK,),
        in_specs=[pl.BlockSpec((_NC, _BLK, D), lambda i: (0, i, 0)),
                  _row_spec(1), _row_spec(1),
                  _full_spec((1, D)), _full_spec((1, D)), _full_spec((1, D))],
        out_specs=_row_spec(D),
        out_shape=jax.ShapeDtypeStruct((Np, D), jnp.float32),
    )(p, d0, d1, b, g, bb)


def _tc_mlp(ht, lin_W, lin_b, lin_g, lin_beta, out_W, out_b):
    T, D = ht.shape
    OUT = out_W.shape[1]
    args = [ht]
    specs = [_full_spec((T, D))]
    for i in range(2):
        args += [lin_W[i], lin_b[i].reshape(1, D), lin_g[i].reshape(1, D),
                 lin_beta[i].reshape(1, D)]
        specs += [_full_spec((D, D)), _full_spec((1, D)),
                  _full_spec((1, D)), _full_spec((1, D))]
    args += [out_W, out_b.reshape(1, OUT)]
    specs += [_full_spec((D, OUT)), _full_spec((1, OUT))]
    return pl.pallas_call(
        _mlp_body,
        grid=(1,),
        in_specs=specs,
        out_specs=_full_spec((T, OUT)),
        out_shape=jax.ShapeDtypeStruct((T, OUT), jnp.float32),
    )(*args)


_F0 = 0.358


def _core_slabs(flat, nbc0, nbc1, pad_val):
    e0 = _NS * nbc0 * _B
    c0 = flat[:e0].reshape(_NS, nbc0, _B)
    c1 = flat[e0:].reshape(_NS, nbc1, _B)
    nbm = max(nbc0, nbc1)
    c0 = jnp.pad(c0, ((0, 0), (0, nbm - nbc0), (0, 0)),
                 constant_values=pad_val)
    c1 = jnp.pad(c1, ((0, 0), (0, nbm - nbc1), (0, 0)),
                 constant_values=pad_val)
    return jnp.stack([c0, c1], axis=1).reshape(_NW, nbm, _B)


def kernel(x, edge_index, target_indices, emb, conv_W, conv_b, conv_g,
           conv_beta, lin_W, lin_b, lin_g, lin_beta, out_W, out_b):
    N = x.shape[0]
    E = edge_index.shape[1]
    V, D = emb.shape
    L_GNN = conv_W.shape[0]

    Np = -(-N // (_NW * _GB)) * (_NW * _GB)
    Ep = -(-E // (_NW * _B)) * (_NW * _B)

    ids_p = jnp.pad(jnp.ravel(x).astype(jnp.int32), (0, Np - N))
    row_p = jnp.pad(edge_index[0].astype(jnp.int32), (0, Ep - E),
                    constant_values=N)
    col_p = jnp.pad(edge_index[1].astype(jnp.int32), (0, Ep - E),
                    constant_values=N)
    pbw = Ep // (_NS * _B)
    nbc0 = max(1, min(pbw - 1, round(pbw * _F0)))
    nbc1 = pbw - nbc0
    row3d = _core_slabs(row_p, nbc0, nbc1, N)
    col3d = _core_slabs(col_p, nbc0, nbc1, N)
    tgt = target_indices.astype(jnp.int32)
    zeros_n = jnp.zeros((Np,), jnp.float32)
    zeros_nd = jnp.zeros((Np, D), jnp.float32)

    h, pdeg0, pdeg1 = _sc_embed_deg(emb, ids_p, col3d, zeros_n, nbc0, nbc1)
    d0 = pdeg0.reshape(Np, 1)
    d1 = pdeg1.reshape(Np, 1)

    zt = _tc_mm_scale(h, conv_W[0], d0, d1)
    h3 = None
    for i in range(L_GNN):
        p = _sc_edge_agg(zt, row3d, col3d, zeros_nd, nbc0, nbc1)
        b = conv_b[i].reshape(1, D)
        g = conv_g[i].reshape(1, D)
        bb = conv_beta[i].reshape(1, D)
        if i + 1 < L_GNN:
            zt = _tc_agg_ln_mm(p, d0, d1, b, g, bb, conv_W[i + 1])
        else:
            h3 = _tc_agg_ln(p, d0, d1, b, g, bb)

    ht = _sc_take(h3, tgt)
    return _tc_mlp(ht, lin_W, lin_b, lin_g, lin_beta, out_W, out_b)


# speedup vs baseline: 1.2448x; 1.2448x over previous
import jax
import jax.numpy as jnp
from jax import lax
from jax.experimental import pallas as pl
from jax.experimental.pallas import tpu as pltpu
from jax.experimental.pallas import tpu_sc as plsc

_NC = 2
_NS = 16
_NW = _NC * _NS
_B = 128
_GB = 80

_MESH = plsc.VectorSubcoreMesh(core_axis_name="c", subcore_axis_name="s")


def _wid():
    return lax.axis_index("s") * _NC + lax.axis_index("c")


def _sc_embed_deg(emb, ids_p, col3d, zeros_n, nbc0, nbc1):
    V, D = emb.shape
    Np = ids_p.shape[0]
    nb_e = col3d.shape[1]
    rows_w = Np // _NW
    rpt = Np // _NS

    def body(emb_h, ids_h, col_h, zn_h, h_out, pdeg0_out, pdeg1_out,
             idx_v, rows_v, cidx_v, ones_v, dacc, sem):
        c = lax.axis_index("c")
        s = lax.axis_index("s")
        w = _wid()
        for k in range(rows_w // _GB):
            base = w * rows_w + k * _GB
            pltpu.sync_copy(ids_h.at[pl.ds(base, _GB)], idx_v)
            pltpu.async_copy(emb_h.at[idx_v], rows_v, sem).wait()
            pltpu.sync_copy(rows_v, h_out.at[pl.ds(base, _GB)])
        t0 = s * rpt
        pltpu.sync_copy(zn_h.at[pl.ds(t0, rpt)], dacc.at[pl.ds(t0, rpt)])
        for j in range(_B // 16):
            ones_v[pl.ds(16 * j, 16)] = jnp.ones((16,), jnp.float32)
        pltpu.sync_copy(col_h.at[w], cidx_v)
        plsc.subcore_barrier()

        def step(j, carry):
            pltpu.sync_copy(ones_v, dacc.at[cidx_v.at[j]], add=True)
            return carry

        @pl.when(c == 0)
        def _():
            lax.fori_loop(0, nbc0, step, 0)

        @pl.when(c != 0)
        def _():
            lax.fori_loop(0, nbc1, step, 0)

        plsc.subcore_barrier()

        @pl.when(c == 0)
        def _():
            pltpu.sync_copy(dacc.at[pl.ds(t0, rpt)],
                            pdeg0_out.at[pl.ds(t0, rpt)])

        @pl.when(c != 0)
        def _():
            pltpu.sync_copy(dacc.at[pl.ds(t0, rpt)],
                            pdeg1_out.at[pl.ds(t0, rpt)])

    f = pl.kernel(
        body,
        out_type=(jax.ShapeDtypeStruct((Np, D), jnp.float32),
                  jax.ShapeDtypeStruct((Np,), jnp.float32),
                  jax.ShapeDtypeStruct((Np,), jnp.float32)),
        mesh=_MESH,
        scratch_types=[
            pltpu.VMEM((_GB,), jnp.int32),
            pltpu.VMEM((_GB, D), jnp.float32),
            pltpu.VMEM((nb_e, _B), jnp.int32),
            pltpu.VMEM((_B,), jnp.float32),
            pltpu.VMEM_SHARED((Np,), jnp.float32),
            pltpu.SemaphoreType.DMA,
        ],
    )
    return f(emb, ids_p, col3d, zeros_n)


def _sc_edge_agg(zt, row3d, col3d, zeros_nd, nbc0, nbc1):
    Np, D = zt.shape
    nb = row3d.shape[1]
    rpt = Np // _NS

    def body(zt_h, row_h, col_h, zn_h, p_out,
             ridx, cidx, rows_v, acc, sem):
        c = lax.axis_index("c")
        s = lax.axis_index("s")
        w = _wid()
        t0 = s * rpt

        @pl.when(c == 0)
        def _():
            pltpu.sync_copy(zt_h.at[pl.ds(t0, rpt)], acc.at[pl.ds(t0, rpt)])

        @pl.when(c != 0)
        def _():
            pltpu.sync_copy(zn_h.at[pl.ds(t0, rpt)], acc.at[pl.ds(t0, rpt)])

        pltpu.sync_copy(row_h.at[w], ridx)
        pltpu.sync_copy(col_h.at[w], cidx)
        plsc.subcore_barrier()

        def step(j, carry):
            pltpu.async_copy(zt_h.at[ridx.at[j]], rows_v, sem).wait()
            pltpu.sync_copy(rows_v, acc.at[cidx.at[j]], add=True)
            return carry

        @pl.when(c == 0)
        def _():
            lax.fori_loop(0, nbc0, step, 0)

        @pl.when(c != 0)
        def _():
            lax.fori_loop(0, nbc1, step, 0)

        plsc.subcore_barrier()
        pltpu.sync_copy(acc.at[pl.ds(t0, rpt)], p_out.at[c, pl.ds(t0, rpt)])

    f = pl.kernel(
        body,
        out_type=jax.ShapeDtypeStruct((_NC, Np, D), jnp.float32),
        mesh=_MESH,
        scratch_types=[
            pltpu.VMEM((nb, _B), jnp.int32),
            pltpu.VMEM((nb, _B), jnp.int32),
            pltpu.VMEM((_B, D), jnp.float32),
            pltpu.VMEM_SHARED((Np, D), jnp.float32),
            pltpu.SemaphoreType.DMA,
        ],
    )
    return f(zt, row3d, col3d, zeros_nd)


def _sc_take(h, tgt):
    Np, D = h.shape
    T = tgt.shape[0]
    rw = T // _NW

    def body(h_h, t_h, o_h, idx_v, rows_v, sem):
        base = _wid() * rw
        pltpu.sync_copy(t_h.at[pl.ds(base, rw)], idx_v)
        pltpu.async_copy(h_h.at[idx_v], rows_v, sem).wait()
        pltpu.sync_copy(rows_v, o_h.at[pl.ds(base, rw)])

    f = pl.kernel(
        body,
        out_type=jax.ShapeDtypeStruct((T, D), jnp.float32),
        mesh=_MESH,
        scratch_types=[
            pltpu.VMEM((rw,), jnp.int32),
            pltpu.VMEM((rw, D), jnp.float32),
            pltpu.SemaphoreType.DMA,
        ],
    )
    return f(h, tgt)


_BLK = 1024


def _dinv_blk(d0_ref, d1_ref):
    deg = d0_ref[...] + d1_ref[...] + 1.0
    return lax.rsqrt(deg)


def _mm_scale_body(h_ref, w_ref, d0_ref, d1_ref, o_ref):
    dinv = _dinv_blk(d0_ref, d1_ref)
    z = jnp.dot(h_ref[...], w_ref[...], preferred_element_type=jnp.float32)
    o_ref[...] = z * dinv


def _ln_relu(t, g, b):
    m = jnp.mean(t, axis=-1, keepdims=True)
    cdev = t - m
    v = jnp.mean(cdev * cdev, axis=-1, keepdims=True)
    return jnp.maximum(cdev * lax.rsqrt(v + 1e-5) * g + b, 0.0)


def _agg_ln_mm_body(p_ref, d0_ref, d1_ref, b_ref, g_ref, bb_ref, w_ref,
                    o_ref):
    dinv = _dinv_blk(d0_ref, d1_ref)
    t = (p_ref[0] + p_ref[1]) * dinv + b_ref[...]
    hr = _ln_relu(t, g_ref[...], bb_ref[...])
    o_ref[...] = jnp.dot(hr, w_ref[...],
                         preferred_element_type=jnp.float32) * dinv


def _agg_ln_body(p_ref, d0_ref, d1_ref, b_ref, g_ref, bb_ref, o_ref):
    dinv = _dinv_blk(d0_ref, d1_ref)
    t = (p_ref[0] + p_ref[1]) * dinv + b_ref[...]
    o_ref[...] = _ln_relu(t, g_ref[...], bb_ref[...])


def _mlp_body(t_ref, w0, b0, g0, bb0, w1, b1, g1, bb1, wo, bo, o_ref):
    t = t_ref[...]
    for w, b, g, bb in ((w0, b0, g0, bb0), (w1, b1, g1, bb1)):
        z = jnp.dot(t, w[...], preferred_element_type=jnp.float32) + b[...]
        t = _ln_relu(z, g[...], bb[...])
    o_ref[...] = jnp.dot(t, wo[...], preferred_element_type=jnp.float32) \
        + bo[...]


def _row_spec(d):
    return pl.BlockSpec((_BLK, d), lambda i: (i, 0))


def _full_spec(shape):
    nd = len(shape)
    return pl.BlockSpec(shape, lambda i, _n=nd: (0,) * _n)


def _tc_mm_scale(h, w, d0, d1):
    Np, D = h.shape
    return pl.pallas_call(
        _mm_scale_body,
        grid=(Np // _BLK,),
        in_specs=[_row_spec(D), _full_spec((D, D)),
                  _row_spec(1), _row_spec(1)],
        out_specs=_row_spec(D),
        out_shape=jax.ShapeDtypeStruct((Np, D), jnp.float32),
    )(h, w, d0, d1)


def _tc_agg_ln_mm(p, d0, d1, b, g, bb, w_next):
    _, Np, D = p.shape
    return pl.pallas_call(
        _agg_ln_mm_body,
        grid=(Np // _BLK,),
        in_specs=[pl.BlockSpec((_NC, _BLK, D), lambda i: (0, i, 0)),
                  _row_spec(1), _row_spec(1),
                  _full_spec((1, D)), _full_spec((1, D)), _full_spec((1, D)),
                  _full_spec((D, D))],
        out_specs=_row_spec(D),
        out_shape=jax.ShapeDtypeStruct((Np, D), jnp.float32),
    )(p, d0, d1, b, g, bb, w_next)


def _tc_agg_ln(p, d0, d1, b, g, bb):
    _, Np, D = p.shape
    return pl.pallas_call(
        _agg_ln_body,
        grid=(Np // _BLK,),
        in_specs=[pl.BlockSpec((_NC, _BLK, D), lambda i: (0, i, 0)),
                  _row_spec(1), _row_spec(1),
                  _full_spec((1, D)), _full_spec((1, D)), _full_spec((1, D))],
        out_specs=_row_spec(D),
        out_shape=jax.ShapeDtypeStruct((Np, D), jnp.float32),
    )(p, d0, d1, b, g, bb)


def _tc_mlp(ht, lin_W, lin_b, lin_g, lin_beta, out_W, out_b):
    T, D = ht.shape
    OUT = out_W.shape[1]
    args = [ht]
    specs = [_full_spec((T, D))]
    for i in range(2):
        args += [lin_W[i], lin_b[i].reshape(1, D), lin_g[i].reshape(1, D),
                 lin_beta[i].reshape(1, D)]
        specs += [_full_spec((D, D)), _full_spec((1, D)),
                  _full_spec((1, D)), _full_spec((1, D))]
    args += [out_W, out_b.reshape(1, OUT)]
    specs += [_full_spec((D, OUT)), _full_spec((1, OUT))]
    return pl.pallas_call(
        _mlp_body,
        grid=(1,),
        in_specs=specs,
        out_specs=_full_spec((T, OUT)),
        out_shape=jax.ShapeDtypeStruct((T, OUT), jnp.float32),
    )(*args)


_F0 = 0.642


def _core_slabs(flat, nbc0, nbc1, pad_val):
    e0 = _NS * nbc0 * _B
    c0 = flat[:e0].reshape(_NS, nbc0, _B)
    c1 = flat[e0:].reshape(_NS, nbc1, _B)
    nbm = max(nbc0, nbc1)
    c0 = jnp.pad(c0, ((0, 0), (0, nbm - nbc0), (0, 0)),
                 constant_values=pad_val)
    c1 = jnp.pad(c1, ((0, 0), (0, nbm - nbc1), (0, 0)),
                 constant_values=pad_val)
    return jnp.stack([c0, c1], axis=1).reshape(_NW, nbm, _B)


def kernel(x, edge_index, target_indices, emb, conv_W, conv_b, conv_g,
           conv_beta, lin_W, lin_b, lin_g, lin_beta, out_W, out_b):
    N = x.shape[0]
    E = edge_index.shape[1]
    V, D = emb.shape
    L_GNN = conv_W.shape[0]

    Np = -(-N // (_NW * _GB)) * (_NW * _GB)
    Ep = -(-E // (_NW * _B)) * (_NW * _B)

    ids_p = jnp.pad(jnp.ravel(x).astype(jnp.int32), (0, Np - N))
    row_p = jnp.pad(edge_index[0].astype(jnp.int32), (0, Ep - E),
                    constant_values=N)
    col_p = jnp.pad(edge_index[1].astype(jnp.int32), (0, Ep - E),
                    constant_values=N)
    pbw = Ep // (_NS * _B)
    nbc0 = max(1, min(pbw - 1, round(pbw * _F0)))
    nbc1 = pbw - nbc0
    row3d = _core_slabs(row_p, nbc0, nbc1, N)
    col3d = _core_slabs(col_p, nbc0, nbc1, N)
    tgt = target_indices.astype(jnp.int32)
    zeros_n = jnp.zeros((Np,), jnp.float32)
    zeros_nd = jnp.zeros((Np, D), jnp.float32)

    h, pdeg0, pdeg1 = _sc_embed_deg(emb, ids_p, col3d, zeros_n, nbc0, nbc1)
    d0 = pdeg0.reshape(Np, 1)
    d1 = pdeg1.reshape(Np, 1)

    zt = _tc_mm_scale(h, conv_W[0], d0, d1)
    h3 = None
    for i in range(L_GNN):
        p = _sc_edge_agg(zt, row3d, col3d, zeros_nd, nbc0, nbc1)
        b = conv_b[i].reshape(1, D)
        g = conv_g[i].reshape(1, D)
        bb = conv_beta[i].reshape(1, D)
        if i + 1 < L_GNN:
            zt = _tc_agg_ln_mm(p, d0, d1, b, g, bb, conv_W[i + 1])
        else:
            h3 = _tc_agg_ln(p, d0, d1, b, g, bb)

    ht = _sc_take(h3, tgt)
    return _tc_mlp(ht, lin_W, lin_b, lin_g, lin_beta, out_W, out_b)


# speedup vs baseline: 1.2708x; 1.0209x over previous
import jax
import jax.numpy as jnp
from jax import lax
from jax.experimental import pallas as pl
from jax.experimental.pallas import tpu as pltpu
from jax.experimental.pallas import tpu_sc as plsc

_NC = 2
_NS = 16
_NW = _NC * _NS
_B = 128
_GB = 80

_MESH = plsc.VectorSubcoreMesh(core_axis_name="c", subcore_axis_name="s")


def _wid():
    return lax.axis_index("s") * _NC + lax.axis_index("c")


def _sc_embed_deg(emb, ids_p, col3d, zeros_n, nbc0, nbc1):
    V, D = emb.shape
    Np = ids_p.shape[0]
    nb_e = col3d.shape[1]
    rows_w = Np // _NW
    rpt = Np // _NS

    def body(emb_h, ids_h, col_h, zn_h, h_out, pdeg0_out, pdeg1_out,
             idx_v, rows_v, cidx_v, ones_v, dacc, sem):
        c = lax.axis_index("c")
        s = lax.axis_index("s")
        w = _wid()
        for k in range(rows_w // _GB):
            base = w * rows_w + k * _GB
            pltpu.sync_copy(ids_h.at[pl.ds(base, _GB)], idx_v)
            pltpu.async_copy(emb_h.at[idx_v], rows_v, sem).wait()
            pltpu.sync_copy(rows_v, h_out.at[pl.ds(base, _GB)])
        t0 = s * rpt
        pltpu.sync_copy(zn_h.at[pl.ds(t0, rpt)], dacc.at[pl.ds(t0, rpt)])
        for j in range(_B // 16):
            ones_v[pl.ds(16 * j, 16)] = jnp.ones((16,), jnp.float32)
        pltpu.sync_copy(col_h.at[w], cidx_v)
        plsc.subcore_barrier()

        def step(j, carry):
            pltpu.sync_copy(ones_v, dacc.at[cidx_v.at[j]], add=True)
            return carry

        @pl.when(c == 0)
        def _():
            lax.fori_loop(0, nbc0, step, 0)

        @pl.when(c != 0)
        def _():
            lax.fori_loop(0, nbc1, step, 0)

        plsc.subcore_barrier()

        @pl.when(c == 0)
        def _():
            pltpu.sync_copy(dacc.at[pl.ds(t0, rpt)],
                            pdeg0_out.at[pl.ds(t0, rpt)])

        @pl.when(c != 0)
        def _():
            pltpu.sync_copy(dacc.at[pl.ds(t0, rpt)],
                            pdeg1_out.at[pl.ds(t0, rpt)])

    f = pl.kernel(
        body,
        out_type=(jax.ShapeDtypeStruct((Np, D), jnp.float32),
                  jax.ShapeDtypeStruct((Np,), jnp.float32),
                  jax.ShapeDtypeStruct((Np,), jnp.float32)),
        mesh=_MESH,
        scratch_types=[
            pltpu.VMEM((_GB,), jnp.int32),
            pltpu.VMEM((_GB, D), jnp.float32),
            pltpu.VMEM((nb_e, _B), jnp.int32),
            pltpu.VMEM((_B,), jnp.float32),
            pltpu.VMEM_SHARED((Np,), jnp.float32),
            pltpu.SemaphoreType.DMA,
        ],
    )
    return f(emb, ids_p, col3d, zeros_n)


def _sc_edge_agg(zt, row3d, col3d, zeros_nd, nbc0, nbc1):
    Np, D = zt.shape
    nb = row3d.shape[1]
    rpt = Np // _NS

    def body(zt_h, row_h, col_h, zn_h, p_out,
             ridx, cidx, rows_v, acc, sem):
        c = lax.axis_index("c")
        s = lax.axis_index("s")
        w = _wid()
        t0 = s * rpt

        @pl.when(c == 0)
        def _():
            pltpu.sync_copy(zt_h.at[pl.ds(t0, rpt)], acc.at[pl.ds(t0, rpt)])

        @pl.when(c != 0)
        def _():
            pltpu.sync_copy(zn_h.at[pl.ds(t0, rpt)], acc.at[pl.ds(t0, rpt)])

        pltpu.sync_copy(row_h.at[w], ridx)
        pltpu.sync_copy(col_h.at[w], cidx)
        plsc.subcore_barrier()

        def step(j, carry):
            pltpu.async_copy(zt_h.at[ridx.at[j]], rows_v, sem).wait()
            pltpu.sync_copy(rows_v, acc.at[cidx.at[j]], add=True)
            return carry

        @pl.when(c == 0)
        def _():
            lax.fori_loop(0, nbc0, step, 0)

        @pl.when(c != 0)
        def _():
            lax.fori_loop(0, nbc1, step, 0)

        plsc.subcore_barrier()
        pltpu.sync_copy(acc.at[pl.ds(t0, rpt)], p_out.at[c, pl.ds(t0, rpt)])

    f = pl.kernel(
        body,
        out_type=jax.ShapeDtypeStruct((_NC, Np, D), jnp.float32),
        mesh=_MESH,
        scratch_types=[
            pltpu.VMEM((nb, _B), jnp.int32),
            pltpu.VMEM((nb, _B), jnp.int32),
            pltpu.VMEM((_B, D), jnp.float32),
            pltpu.VMEM_SHARED((Np, D), jnp.float32),
            pltpu.SemaphoreType.DMA,
        ],
    )
    return f(zt, row3d, col3d, zeros_nd)


def _sc_take(h, tgt):
    Np, D = h.shape
    T = tgt.shape[0]
    rw = T // _NW

    def body(h_h, t_h, o_h, idx_v, rows_v, sem):
        base = _wid() * rw
        pltpu.sync_copy(t_h.at[pl.ds(base, rw)], idx_v)
        pltpu.async_copy(h_h.at[idx_v], rows_v, sem).wait()
        pltpu.sync_copy(rows_v, o_h.at[pl.ds(base, rw)])

    f = pl.kernel(
        body,
        out_type=jax.ShapeDtypeStruct((T, D), jnp.float32),
        mesh=_MESH,
        scratch_types=[
            pltpu.VMEM((rw,), jnp.int32),
            pltpu.VMEM((rw, D), jnp.float32),
            pltpu.SemaphoreType.DMA,
        ],
    )
    return f(h, tgt)


_BLK = 1024


def _dinv_blk(d0_ref, d1_ref):
    deg = d0_ref[...] + d1_ref[...] + 1.0
    return lax.rsqrt(deg)


def _mm_scale_body(h_ref, w_ref, d0_ref, d1_ref, o_ref):
    dinv = _dinv_blk(d0_ref, d1_ref)
    z = jnp.dot(h_ref[...], w_ref[...], preferred_element_type=jnp.float32)
    o_ref[...] = z * dinv


def _ln_relu(t, g, b):
    m = jnp.mean(t, axis=-1, keepdims=True)
    cdev = t - m
    v = jnp.mean(cdev * cdev, axis=-1, keepdims=True)
    return jnp.maximum(cdev * lax.rsqrt(v + 1e-5) * g + b, 0.0)


def _agg_ln_mm_body(p_ref, d0_ref, d1_ref, b_ref, g_ref, bb_ref, w_ref,
                    o_ref):
    dinv = _dinv_blk(d0_ref, d1_ref)
    t = (p_ref[0] + p_ref[1]) * dinv + b_ref[...]
    hr = _ln_relu(t, g_ref[...], bb_ref[...])
    o_ref[...] = jnp.dot(hr, w_ref[...],
                         preferred_element_type=jnp.float32) * dinv


def _agg_ln_body(p_ref, d0_ref, d1_ref, b_ref, g_ref, bb_ref, o_ref):
    dinv = _dinv_blk(d0_ref, d1_ref)
    t = (p_ref[0] + p_ref[1]) * dinv + b_ref[...]
    o_ref[...] = _ln_relu(t, g_ref[...], bb_ref[...])


def _mlp_body(t_ref, w0, b0, g0, bb0, w1, b1, g1, bb1, wo, bo, o_ref):
    t = t_ref[...]
    for w, b, g, bb in ((w0, b0, g0, bb0), (w1, b1, g1, bb1)):
        z = jnp.dot(t, w[...], preferred_element_type=jnp.float32) + b[...]
        t = _ln_relu(z, g[...], bb[...])
    o_ref[...] = jnp.dot(t, wo[...], preferred_element_type=jnp.float32) \
        + bo[...]


def _row_spec(d):
    return pl.BlockSpec((_BLK, d), lambda i: (i, 0))


def _full_spec(shape):
    nd = len(shape)
    return pl.BlockSpec(shape, lambda i, _n=nd: (0,) * _n)


def _tc_mm_scale(h, w, d0, d1):
    Np, D = h.shape
    return pl.pallas_call(
        _mm_scale_body,
        grid=(Np // _BLK,),
        in_specs=[_row_spec(D), _full_spec((D, D)),
                  _row_spec(1), _row_spec(1)],
        out_specs=_row_spec(D),
        out_shape=jax.ShapeDtypeStruct((Np, D), jnp.float32),
    )(h, w, d0, d1)


def _tc_agg_ln_mm(p, d0, d1, b, g, bb, w_next):
    _, Np, D = p.shape
    return pl.pallas_call(
        _agg_ln_mm_body,
        grid=(Np // _BLK,),
        in_specs=[pl.BlockSpec((_NC, _BLK, D), lambda i: (0, i, 0)),
                  _row_spec(1), _row_spec(1),
                  _full_spec((1, D)), _full_spec((1, D)), _full_spec((1, D)),
                  _full_spec((D, D))],
        out_specs=_row_spec(D),
        out_shape=jax.ShapeDtypeStruct((Np, D), jnp.float32),
    )(p, d0, d1, b, g, bb, w_next)


def _tc_agg_ln(p, d0, d1, b, g, bb):
    _, Np, D = p.shape
    return pl.pallas_call(
        _agg_ln_body,
        grid=(Np // _BLK,),
        in_specs=[pl.BlockSpec((_NC, _BLK, D), lambda i: (0, i, 0)),
                  _row_spec(1), _row_spec(1),
                  _full_spec((1, D)), _full_spec((1, D)), _full_spec((1, D))],
        out_specs=_row_spec(D),
        out_shape=jax.ShapeDtypeStruct((Np, D), jnp.float32),
    )(p, d0, d1, b, g, bb)


def _tc_mlp(ht, lin_W, lin_b, lin_g, lin_beta, out_W, out_b):
    T, D = ht.shape
    OUT = out_W.shape[1]
    args = [ht]
    specs = [_full_spec((T, D))]
    for i in range(2):
        args += [lin_W[i], lin_b[i].reshape(1, D), lin_g[i].reshape(1, D),
                 lin_beta[i].reshape(1, D)]
        specs += [_full_spec((D, D)), _full_spec((1, D)),
                  _full_spec((1, D)), _full_spec((1, D))]
    args += [out_W, out_b.reshape(1, OUT)]
    specs += [_full_spec((D, OUT)), _full_spec((1, OUT))]
    return pl.pallas_call(
        _mlp_body,
        grid=(1,),
        in_specs=specs,
        out_specs=_full_spec((T, OUT)),
        out_shape=jax.ShapeDtypeStruct((T, OUT), jnp.float32),
    )(*args)


_F0 = 0.706


def _core_slabs(flat, nbc0, nbc1, pad_val):
    e0 = _NS * nbc0 * _B
    c0 = flat[:e0].reshape(_NS, nbc0, _B)
    c1 = flat[e0:].reshape(_NS, nbc1, _B)
    nbm = max(nbc0, nbc1)
    c0 = jnp.pad(c0, ((0, 0), (0, nbm - nbc0), (0, 0)),
                 constant_values=pad_val)
    c1 = jnp.pad(c1, ((0, 0), (0, nbm - nbc1), (0, 0)),
                 constant_values=pad_val)
    return jnp.stack([c0, c1], axis=1).reshape(_NW, nbm, _B)


def kernel(x, edge_index, target_indices, emb, conv_W, conv_b, conv_g,
           conv_beta, lin_W, lin_b, lin_g, lin_beta, out_W, out_b):
    N = x.shape[0]
    E = edge_index.shape[1]
    V, D = emb.shape
    L_GNN = conv_W.shape[0]

    Np = -(-N // (_NW * _GB)) * (_NW * _GB)
    Ep = -(-E // (_NW * _B)) * (_NW * _B)

    ids_p = jnp.pad(jnp.ravel(x).astype(jnp.int32), (0, Np - N))
    row_p = jnp.pad(edge_index[0].astype(jnp.int32), (0, Ep - E),
                    constant_values=N)
    col_p = jnp.pad(edge_index[1].astype(jnp.int32), (0, Ep - E),
                    constant_values=N)
    pbw = Ep // (_NS * _B)
    nbc0 = max(1, min(pbw - 1, round(pbw * _F0)))
    nbc1 = pbw - nbc0
    row3d = _core_slabs(row_p, nbc0, nbc1, N)
    col3d = _core_slabs(col_p, nbc0, nbc1, N)
    tgt = target_indices.astype(jnp.int32)
    zeros_n = jnp.zeros((Np,), jnp.float32)
    zeros_nd = jnp.zeros((Np, D), jnp.float32)

    h, pdeg0, pdeg1 = _sc_embed_deg(emb, ids_p, col3d, zeros_n, nbc0, nbc1)
    d0 = pdeg0.reshape(Np, 1)
    d1 = pdeg1.reshape(Np, 1)

    zt = _tc_mm_scale(h, conv_W[0], d0, d1)
    h3 = None
    for i in range(L_GNN):
        p = _sc_edge_agg(zt, row3d, col3d, zeros_nd, nbc0, nbc1)
        b = conv_b[i].reshape(1, D)
        g = conv_g[i].reshape(1, D)
        bb = conv_beta[i].reshape(1, D)
        if i + 1 < L_GNN:
            zt = _tc_agg_ln_mm(p, d0, d1, b, g, bb, conv_W[i + 1])
        else:
            h3 = _tc_agg_ln(p, d0, d1, b, g, bb)

    ht = _sc_take(h3, tgt)
    return _tc_mlp(ht, lin_W, lin_b, lin_g, lin_beta, out_W, out_b)
